# baseline (speedup 1.0000x reference)
import functools

import jax
import jax.numpy as jnp
from jax import lax
from jax.experimental import pallas as pl
from jax.experimental.pallas import tpu as pltpu
from jax.experimental.pallas import tpu_sc as plsc

_N = 10000
_D = 128
_E = 320000
_G = 256
_NC = 2
_NS = 16
_TILES = _NC * _NS
_EPT = _E // _TILES
_CH = 128
_NCHUNK = _EPT // _CH
_TAIL = _EPT - _NCHUNK * _CH
_NBUF = 2
_NACC = 10112
_STRIPE = _NACC // _NS
_NDEG = 10240
_DSTRIPE = _NDEG // _NS
_NBLK = 1000
_NGRID = _N // _NBLK

_f32 = jnp.float32


def _zero_vmem_2d(ref, rows):
    def body(i, _):
        r = i // 8
        l = (i % 8) * 16
        ref[r, pl.ds(l, 16)] = jnp.zeros((16,), _f32)
        return 0
    lax.fori_loop(0, rows * 8, body, 0)


@functools.cache
def _get_deg_kernel():
    mesh = plsc.VectorSubcoreMesh(
        core_axis_name="c", subcore_axis_name="s",
        num_cores=_NC, num_subcores=_NS)
    return pl.kernel(
        _deg_body,
        out_type=jax.ShapeDtypeStruct((_NC * _NDEG,), _f32),
        mesh=mesh,
        scratch_types=[
            pltpu.MemorySpace.VMEM_SHARED((_NDEG,), _f32),
            pltpu.MemorySpace.VMEM((_NBUF, _CH), jnp.int32),
            pltpu.MemorySpace.VMEM((1, _TAIL), jnp.int32),
            pltpu.MemorySpace.VMEM((_CH,), _f32),
            pltpu.MemorySpace.VMEM((_DSTRIPE,), _f32),
            [pltpu.SemaphoreType.DMA for _ in range(_NBUF)],
        ],
    )


def _deg_body(dst_hbm, out_hbm, acc_sh, dstb, dstt, ones_v, zrow_v, dsems):
    c = lax.axis_index("c")
    s = lax.axis_index("s")
    base = (c * _NS + s) * _EPT

    def zb(i, _):
        zrow_v[pl.ds(i * 16, 16)] = jnp.zeros((16,), _f32)
        return 0
    lax.fori_loop(0, _DSTRIPE // 16, zb, 0)

    def ob(i, _):
        ones_v[pl.ds(i * 16, 16)] = jnp.ones((16,), _f32)
        return 0
    lax.fori_loop(0, _CH // 16, ob, 0)

    for b in range(_NBUF):
        pltpu.async_copy(dst_hbm.at[pl.ds(base + b * _CH, _CH)],
                         dstb.at[b], dsems[b])
    pltpu.sync_copy(zrow_v, acc_sh.at[pl.ds(s * _DSTRIPE, _DSTRIPE)])
    plsc.subcore_barrier()

    def outer(i, _):
        j0 = i * _NBUF
        for b in range(_NBUF):
            j = j0 + b
            pltpu.make_async_copy(
                dst_hbm.at[pl.ds(base + b * _CH, _CH)],
                dstb.at[b], dsems[b]).wait()
            pltpu.sync_copy(ones_v, acc_sh.at[dstb.at[b]], add=True)
            jn = jnp.minimum(j + _NBUF, _NCHUNK - 1)
            off = pl.multiple_of(base + jn * _CH, 8)
            pltpu.async_copy(dst_hbm.at[pl.ds(off, _CH)], dstb.at[b],
                             dsems[b])
        return 0
    lax.fori_loop(0, _NCHUNK // _NBUF, outer, 0)

    for b in range(_NBUF):
        pltpu.make_async_copy(dst_hbm.at[pl.ds(base, _CH)],
                              dstb.at[b], dsems[b]).wait()
    pltpu.sync_copy(dst_hbm.at[pl.ds(base + _NCHUNK * _CH, _TAIL)],
                    dstt.at[0])
    pltpu.sync_copy(ones_v.at[pl.ds(0, _TAIL)], acc_sh.at[dstt.at[0]],
                    add=True)

    plsc.subcore_barrier()
    pltpu.sync_copy(acc_sh.at[pl.ds(s * _DSTRIPE, _DSTRIPE)],
                    out_hbm.at[pl.ds(c * _NDEG + s * _DSTRIPE, _DSTRIPE)])


@functools.cache
def _get_scatter_kernel():
    mesh = plsc.VectorSubcoreMesh(
        core_axis_name="c", subcore_axis_name="s",
        num_cores=_NC, num_subcores=_NS)
    return pl.kernel(
        _scatter_body,
        out_type=[jax.ShapeDtypeStruct((_NACC, _D), _f32),
                  jax.ShapeDtypeStruct((_NACC, _D), _f32)],
        mesh=mesh,
        scratch_types=[
            pltpu.MemorySpace.VMEM_SHARED((_NACC, _D), _f32),
            pltpu.MemorySpace.VMEM((_EPT,), jnp.int32),
            pltpu.MemorySpace.VMEM((_NBUF, _CH), jnp.int32),
            pltpu.MemorySpace.VMEM((1, _TAIL), jnp.int32),
            pltpu.MemorySpace.VMEM((_TAIL, _D), _f32),
            [pltpu.MemorySpace.VMEM((_CH, _D), _f32) for _ in range(_NBUF)],
            [pltpu.SemaphoreType.DMA for _ in range(_NBUF)],
            [pltpu.SemaphoreType.DMA for _ in range(_NBUF)],
        ],
    )


def _scatter_body(g_hbm, src_hbm, dst_hbm, out0_hbm, out1_hbm,
                  acc_sh, src_all, dstb, dstt, buft, bufs, gsems, dsems):
    c = lax.axis_index("c")
    s = lax.axis_index("s")
    base = (c * _NS + s) * _EPT

    pltpu.sync_copy(src_hbm.at[pl.ds(base, _EPT)], src_all)

    with jax.named_scope("zero_acc"):
        _zero_vmem_2d(bufs[0], _CH)
        for k in range(_STRIPE // _CH):
            pltpu.sync_copy(bufs[0],
                            acc_sh.at[pl.ds(s * _STRIPE + k * _CH, _CH)])
        rem = _STRIPE - (_STRIPE // _CH) * _CH
        if rem:
            pltpu.sync_copy(
                bufs[0].at[pl.ds(0, rem)],
                acc_sh.at[pl.ds(s * _STRIPE + (_STRIPE // _CH) * _CH, rem)])

    for b in range(_NBUF):
        pltpu.async_copy(dst_hbm.at[pl.ds(base + b * _CH, _CH)],
                         dstb.at[b], dsems[b])
        pltpu.async_copy(g_hbm.at[src_all.at[pl.ds(b * _CH, _CH)]],
                         bufs[b], gsems[b])
    plsc.subcore_barrier()

    with jax.named_scope("edge_loop"):
        def outer(i, _):
            j0 = i * _NBUF
            for b in range(_NBUF):
                j = j0 + b
                pltpu.make_async_copy(g_hbm.at[src_all.at[pl.ds(0, _CH)]],
                                      bufs[b], gsems[b]).wait()
                pltpu.make_async_copy(dst_hbm.at[pl.ds(base, _CH)],
                                      dstb.at[b], dsems[b]).wait()
                pltpu.sync_copy(bufs[b], acc_sh.at[dstb.at[b]], add=True)
                jn = jnp.minimum(j + _NBUF, _NCHUNK - 1)
                goff = pl.multiple_of(jn * _CH, 8)
                doff = pl.multiple_of(base + jn * _CH, 8)
                pltpu.async_copy(dst_hbm.at[pl.ds(doff, _CH)], dstb.at[b],
                                 dsems[b])
                pltpu.async_copy(g_hbm.at[src_all.at[pl.ds(goff, _CH)]],
                                 bufs[b], gsems[b])
            return 0
        lax.fori_loop(0, _NCHUNK // _NBUF, outer, 0)

        for b in range(_NBUF):
            pltpu.make_async_copy(g_hbm.at[src_all.at[pl.ds(0, _CH)]],
                                  bufs[b], gsems[b]).wait()
            pltpu.make_async_copy(dst_hbm.at[pl.ds(base, _CH)],
                                  dstb.at[b], dsems[b]).wait()

        pltpu.sync_copy(dst_hbm.at[pl.ds(base + _NCHUNK * _CH, _TAIL)],
                        dstt.at[0])
        pltpu.async_copy(
            g_hbm.at[src_all.at[pl.ds(_NCHUNK * _CH, _TAIL)]],
            buft, gsems[0]).wait()
        pltpu.sync_copy(buft, acc_sh.at[dstt.at[0]], add=True)

    with jax.named_scope("end_barrier"):
        plsc.subcore_barrier()
    with jax.named_scope("copy_out"):
        @pl.when(c == 0)
        def _():
            pltpu.sync_copy(acc_sh.at[pl.ds(s * _STRIPE, _STRIPE)],
                            out0_hbm.at[pl.ds(s * _STRIPE, _STRIPE)])

        @pl.when(c == 1)
        def _():
            pltpu.sync_copy(acc_sh.at[pl.ds(s * _STRIPE, _STRIPE)],
                            out1_hbm.at[pl.ds(s * _STRIPE, _STRIPE)])


def _k1a_body(x_ref, W1_ref, o_ref):
    o_ref[...] = jnp.dot(x_ref[...], W1_ref[...],
                         preferred_element_type=_f32)


def _k1b_body(p_ref, degT_ref, o_ref):
    dinv = lax.rsqrt(degT_ref[:, 0:1] + degT_ref[:, 1:2] + 1.0)
    o_ref[...] = p_ref[...] * dinv


def _k3_body(acc_a_ref, acc_b_ref, g1_ref, degT_ref, b1_ref, W2_ref, o_ref):
    dinv = lax.rsqrt(degT_ref[:, 0:1] + degT_ref[:, 1:2] + 1.0)
    h1 = jax.nn.relu((acc_a_ref[...] + acc_b_ref[...] + g1_ref[...]) * dinv
                     + b1_ref[...])
    o_ref[...] = jnp.dot(h1, W2_ref[...], preferred_element_type=_f32) * dinv


def _k5_body(acc_a_ref, acc_b_ref, g2_ref, degT_ref, b2_ref, batch_ref,
             axp_ref, Wl1_ref, bl1_ref, Wax_ref, bax_ref,
             Wl2h_ref, Wl2a_ref, bl2_ref, o_ref, sums_acc, cnts_acc):
    i = pl.program_id(0)

    @pl.when(i == 0)
    def _():
        sums_acc[...] = jnp.zeros_like(sums_acc)
        cnts_acc[...] = jnp.zeros_like(cnts_acc)

    dinv = lax.rsqrt(degT_ref[:, 0:1] + degT_ref[:, 1:2] + 1.0)
    h2 = jax.nn.relu((acc_a_ref[...] + acc_b_ref[...] + g2_ref[...]) * dinv
                     + b2_ref[...])
    gid = lax.broadcasted_iota(jnp.int32, (_NBLK, _G), 1)
    oh = jnp.where(batch_ref[...] == gid, 1.0, 0.0)
    sums_acc[...] += lax.dot_general(
        oh, h2, (((0,), (0,)), ((), ())), preferred_element_type=_f32)
    cnts_acc[...] += lax.dot_general(
        oh, jnp.ones((_NBLK, _D), _f32), (((0,), (0,)), ((), ())),
        preferred_element_type=_f32)

    @pl.when(i == _NGRID - 1)
    def _():
        mean = sums_acc[...] / jnp.maximum(cnts_acc[...], 1.0)
        ho = jnp.dot(mean, Wl1_ref[...],
                     preferred_element_type=_f32) + bl1_ref[...]
        ax = jnp.dot(axp_ref[...], Wax_ref[...],
                     preferred_element_type=_f32) + bax_ref[...]
        o_ref[...] = (jnp.dot(ho, Wl2h_ref[...], preferred_element_type=_f32)
                      + jnp.dot(ax, Wl2a_ref[...],
                                preferred_element_type=_f32)
                      + bl2_ref[...])


def _deg_call(dstp):
    return _get_deg_kernel()(dstp)


def _scatter_call(g, srcp, dstp):
    return _get_scatter_kernel()(g, srcp, dstp)


_spec_full128 = pl.BlockSpec((_D, _D), lambda i: (0, 0))
_spec_row128 = pl.BlockSpec((1, _D), lambda i: (0, 0))
_spec_nblk = pl.BlockSpec((_NBLK, _D), lambda i: (i, 0))
_spec_deg = pl.BlockSpec((_NBLK, 2), lambda i: (i, 0))


def _k1a_call(x, W1):
    return pl.pallas_call(
        _k1a_body,
        grid=(_NGRID,),
        in_specs=[_spec_nblk, _spec_full128],
        out_specs=_spec_nblk,
        out_shape=jax.ShapeDtypeStruct((_N, _D), _f32),
    )(x, W1)


def _k1b_call(p1, degT):
    return pl.pallas_call(
        _k1b_body,
        grid=(_NGRID,),
        in_specs=[_spec_nblk, _spec_deg],
        out_specs=_spec_nblk,
        out_shape=jax.ShapeDtypeStruct((_N, _D), _f32),
    )(p1, degT)


def _k3_call(acc0, acc1, g1, degT, b1p, W2):
    return pl.pallas_call(
        _k3_body,
        grid=(_NGRID,),
        in_specs=[_spec_nblk, _spec_nblk, _spec_nblk, _spec_deg,
                  _spec_row128, _spec_full128],
        out_specs=_spec_nblk,
        out_shape=jax.ShapeDtypeStruct((_N, _D), _f32),
    )(acc0, acc1, g1, degT, b1p, W2)


def _k5_call(acc0, acc1, g2, degT, b2p, batch2d, axp, Wl1, bl1p, Waxp, baxp,
             Wl2h, Wl2a, bl2p):
    return pl.pallas_call(
        _k5_body,
        grid=(_NGRID,),
        in_specs=[_spec_nblk, _spec_nblk, _spec_nblk, _spec_deg,
                  _spec_row128,
                  pl.BlockSpec((_NBLK, 1), lambda i: (i, 0)),
                  pl.BlockSpec((_G, _D), lambda i: (0, 0)),
                  _spec_full128, _spec_row128, _spec_full128, _spec_row128,
                  _spec_full128, _spec_full128, _spec_row128],
        out_specs=pl.BlockSpec((_G, _D), lambda i: (0, 0)),
        out_shape=jax.ShapeDtypeStruct((_G, _D), _f32),
        scratch_shapes=[
            pltpu.VMEM((_G, _D), _f32),
            pltpu.VMEM((_G, _D), _f32),
        ],
    )(acc0, acc1, g2, degT, b2p, batch2d, axp, Wl1, bl1p, Waxp, baxp,
      Wl2h, Wl2a, bl2p)


def kernel(x, edge_index, batch, ax_data, W1, b1, W2, b2, Wl1, bl1,
           Wax, bax, Wl2, bl2):
    dstp = edge_index[1]
    srcp = lax.optimization_barrier(edge_index[0])
    batch2d = batch.reshape(_N, 1)
    b1p = b1.reshape(1, _D)
    b2p = b2.reshape(1, _D)
    axp = jnp.pad(ax_data, ((0, 0), (0, _D - ax_data.shape[1])))
    Waxp = jnp.pad(Wax, ((0, _D - Wax.shape[0]), (0, _D - Wax.shape[1])))
    baxp = jnp.pad(bax, (0, _D - bax.shape[0])).reshape(1, _D)
    Wl2h = jnp.pad(Wl2[:_D], ((0, 0), (0, _D - Wl2.shape[1])))
    Wl2a = jnp.pad(Wl2[_D:], ((0, _D - (Wl2.shape[0] - _D)),
                              (0, _D - Wl2.shape[1])))
    bl2p = jnp.pad(bl2, (0, _D - bl2.shape[0])).reshape(1, _D)
    bl1p = bl1.reshape(1, _D)

    deg2 = _deg_call(dstp)
    p1 = _k1a_call(x, W1)
    degT = deg2.reshape(_NC, _NDEG).T
    g1 = _k1b_call(p1, degT)
    acc1a, acc1b = _scatter_call(g1, srcp, dstp)
    g2 = _k3_call(acc1a, acc1b, g1, degT, b1p, W2)
    acc2a, acc2b = _scatter_call(g2, srcp, dstp)
    out128 = _k5_call(acc2a, acc2b, g2, degT, b2p, batch2d, axp, Wl1, bl1p,
                      Waxp, baxp, Wl2h, Wl2a, bl2p)
    return out128[:, :Wl2.shape[1]]


# speedup vs baseline: 1.0015x; 1.0015x over previous
import functools

import jax
import jax.numpy as jnp
from jax import lax
from jax.experimental import pallas as pl
from jax.experimental.pallas import tpu as pltpu
from jax.experimental.pallas import tpu_sc as plsc

_N = 10000
_D = 128
_E = 320000
_G = 256
_NC = 2
_NS = 16
_TILES = _NC * _NS
_EPT = _E // _TILES
_CH = 128
_NCHUNK = _EPT // _CH
_TAIL = _EPT - _NCHUNK * _CH
_NBUF = 2
_NACC = 10112
_STRIPE = _NACC // _NS
_NDEG = 10240
_DSTRIPE = _NDEG // _NS
_NBLK = 1000
_NGRID = _N // _NBLK

_f32 = jnp.float32


def _zero_vmem_2d(ref, rows):
    def body(i, _):
        r = i // 8
        l = (i % 8) * 16
        ref[r, pl.ds(l, 16)] = jnp.zeros((16,), _f32)
        return 0
    lax.fori_loop(0, rows * 8, body, 0)


@functools.cache
def _get_deg_kernel():
    mesh = plsc.VectorSubcoreMesh(
        core_axis_name="c", subcore_axis_name="s",
        num_cores=_NC, num_subcores=_NS)
    return pl.kernel(
        _deg_body,
        out_type=jax.ShapeDtypeStruct((_NC * _NDEG,), _f32),
        mesh=mesh,
        scratch_types=[
            pltpu.MemorySpace.VMEM_SHARED((_NDEG,), _f32),
            pltpu.MemorySpace.VMEM((_NBUF, _CH), jnp.int32),
            pltpu.MemorySpace.VMEM((1, _TAIL), jnp.int32),
            pltpu.MemorySpace.VMEM((_CH,), _f32),
            pltpu.MemorySpace.VMEM((_DSTRIPE,), _f32),
            [pltpu.SemaphoreType.DMA for _ in range(_NBUF)],
        ],
    )


def _deg_body(dst_hbm, out_hbm, acc_sh, dstb, dstt, ones_v, zrow_v, dsems):
    c = lax.axis_index("c")
    s = lax.axis_index("s")
    base = (c * _NS + s) * _EPT

    def zb(i, _):
        zrow_v[pl.ds(i * 16, 16)] = jnp.zeros((16,), _f32)
        return 0
    lax.fori_loop(0, _DSTRIPE // 16, zb, 0)

    def ob(i, _):
        ones_v[pl.ds(i * 16, 16)] = jnp.ones((16,), _f32)
        return 0
    lax.fori_loop(0, _CH // 16, ob, 0)

    for b in range(_NBUF):
        pltpu.async_copy(dst_hbm.at[pl.ds(base + b * _CH, _CH)],
                         dstb.at[b], dsems[b])
    pltpu.sync_copy(zrow_v, acc_sh.at[pl.ds(s * _DSTRIPE, _DSTRIPE)])
    plsc.subcore_barrier()

    def outer(i, _):
        j0 = i * _NBUF
        for b in range(_NBUF):
            j = j0 + b
            pltpu.make_async_copy(
                dst_hbm.at[pl.ds(base + b * _CH, _CH)],
                dstb.at[b], dsems[b]).wait()
            pltpu.sync_copy(ones_v, acc_sh.at[dstb.at[b]], add=True)
            jn = jnp.minimum(j + _NBUF, _NCHUNK - 1)
            off = pl.multiple_of(base + jn * _CH, 8)
            pltpu.async_copy(dst_hbm.at[pl.ds(off, _CH)], dstb.at[b],
                             dsems[b])
        return 0
    lax.fori_loop(0, _NCHUNK // _NBUF, outer, 0)

    for b in range(_NBUF):
        pltpu.make_async_copy(dst_hbm.at[pl.ds(base, _CH)],
                              dstb.at[b], dsems[b]).wait()
    pltpu.sync_copy(dst_hbm.at[pl.ds(base + _NCHUNK * _CH, _TAIL)],
                    dstt.at[0])
    pltpu.sync_copy(ones_v.at[pl.ds(0, _TAIL)], acc_sh.at[dstt.at[0]],
                    add=True)

    plsc.subcore_barrier()
    pltpu.sync_copy(acc_sh.at[pl.ds(s * _DSTRIPE, _DSTRIPE)],
                    out_hbm.at[pl.ds(c * _NDEG + s * _DSTRIPE, _DSTRIPE)])


@functools.cache
def _get_scatter_kernel():
    mesh = plsc.VectorSubcoreMesh(
        core_axis_name="c", subcore_axis_name="s",
        num_cores=_NC, num_subcores=_NS)
    return pl.kernel(
        _scatter_body,
        out_type=[jax.ShapeDtypeStruct((_NACC, _D), _f32),
                  jax.ShapeDtypeStruct((_NACC, _D), _f32)],
        mesh=mesh,
        scratch_types=[
            pltpu.MemorySpace.VMEM_SHARED((_NACC, _D), _f32),
            pltpu.MemorySpace.VMEM((_EPT,), jnp.int32),
            pltpu.MemorySpace.VMEM((_NBUF, _CH), jnp.int32),
            pltpu.MemorySpace.VMEM((1, _TAIL), jnp.int32),
            pltpu.MemorySpace.VMEM((_TAIL, _D), _f32),
            [pltpu.MemorySpace.VMEM((_CH, _D), _f32) for _ in range(_NBUF)],
            [pltpu.SemaphoreType.DMA for _ in range(_NBUF)],
            [pltpu.SemaphoreType.DMA for _ in range(_NBUF)],
        ],
    )


def _scatter_body(g_hbm, src_hbm, dst_hbm, out0_hbm, out1_hbm,
                  acc_sh, src_all, dstb, dstt, buft, bufs, gsems, dsems):
    c = lax.axis_index("c")
    s = lax.axis_index("s")
    base = (c * _NS + s) * _EPT

    pltpu.sync_copy(src_hbm.at[pl.ds(base, _EPT)], src_all)

    with jax.named_scope("zero_acc"):
        _zero_vmem_2d(bufs[0], _CH)
        for k in range(_STRIPE // _CH):
            pltpu.sync_copy(bufs[0],
                            acc_sh.at[pl.ds(s * _STRIPE + k * _CH, _CH)])
        rem = _STRIPE - (_STRIPE // _CH) * _CH
        if rem:
            pltpu.sync_copy(
                bufs[0].at[pl.ds(0, rem)],
                acc_sh.at[pl.ds(s * _STRIPE + (_STRIPE // _CH) * _CH, rem)])

    for b in range(_NBUF):
        pltpu.async_copy(dst_hbm.at[pl.ds(base + b * _CH, _CH)],
                         dstb.at[b], dsems[b])
        pltpu.async_copy(g_hbm.at[src_all.at[pl.ds(b * _CH, _CH)]],
                         bufs[b], gsems[b])
    plsc.subcore_barrier()

    with jax.named_scope("edge_loop"):
        def outer(i, _):
            j0 = i * _NBUF
            for b in range(_NBUF):
                j = j0 + b
                pltpu.make_async_copy(g_hbm.at[src_all.at[pl.ds(0, _CH)]],
                                      bufs[b], gsems[b]).wait()
                pltpu.make_async_copy(dst_hbm.at[pl.ds(base, _CH)],
                                      dstb.at[b], dsems[b]).wait()
                pltpu.sync_copy(bufs[b], acc_sh.at[dstb.at[b]], add=True)
                jn = jnp.minimum(j + _NBUF, _NCHUNK - 1)
                goff = pl.multiple_of(jn * _CH, 8)
                doff = pl.multiple_of(base + jn * _CH, 8)
                pltpu.async_copy(dst_hbm.at[pl.ds(doff, _CH)], dstb.at[b],
                                 dsems[b])
                pltpu.async_copy(g_hbm.at[src_all.at[pl.ds(goff, _CH)]],
                                 bufs[b], gsems[b])
            return 0
        lax.fori_loop(0, _NCHUNK // _NBUF, outer, 0)

        for b in range(_NBUF):
            pltpu.make_async_copy(g_hbm.at[src_all.at[pl.ds(0, _CH)]],
                                  bufs[b], gsems[b]).wait()
            pltpu.make_async_copy(dst_hbm.at[pl.ds(base, _CH)],
                                  dstb.at[b], dsems[b]).wait()

        pltpu.sync_copy(dst_hbm.at[pl.ds(base + _NCHUNK * _CH, _TAIL)],
                        dstt.at[0])
        pltpu.async_copy(
            g_hbm.at[src_all.at[pl.ds(_NCHUNK * _CH, _TAIL)]],
            buft, gsems[0]).wait()
        pltpu.sync_copy(buft, acc_sh.at[dstt.at[0]], add=True)

    with jax.named_scope("end_barrier"):
        plsc.subcore_barrier()
    with jax.named_scope("copy_out"):
        @pl.when(c == 0)
        def _():
            pltpu.sync_copy(acc_sh.at[pl.ds(s * _STRIPE, _STRIPE)],
                            out0_hbm.at[pl.ds(s * _STRIPE, _STRIPE)])

        @pl.when(c == 1)
        def _():
            pltpu.sync_copy(acc_sh.at[pl.ds(s * _STRIPE, _STRIPE)],
                            out1_hbm.at[pl.ds(s * _STRIPE, _STRIPE)])


def _k1a_body(x_ref, W1_ref, o_ref):
    o_ref[...] = jnp.dot(x_ref[...], W1_ref[...],
                         preferred_element_type=_f32)


def _k1b_body(p_ref, degT_ref, o_ref):
    dinv = lax.rsqrt(degT_ref[:, 0:1] + degT_ref[:, 1:2] + 1.0)
    o_ref[...] = p_ref[...] * dinv


def _k3_body(acc_a_ref, acc_b_ref, g1_ref, degT_ref, b1_ref, W2_ref, o_ref):
    dinv = lax.rsqrt(degT_ref[:, 0:1] + degT_ref[:, 1:2] + 1.0)
    h1 = jax.nn.relu((acc_a_ref[...] + acc_b_ref[...] + g1_ref[...]) * dinv
                     + b1_ref[...])
    o_ref[...] = jnp.dot(h1, W2_ref[...], preferred_element_type=_f32) * dinv


def _k5_body(acc_a_ref, acc_b_ref, g2_ref, degT_ref, b2_ref, batch_ref,
             axp_ref, Wl1_ref, bl1_ref, Wax_ref, bax_ref,
             Wl2h_ref, Wl2a_ref, bl2_ref, o_ref, sums_acc, cnts_acc):
    i = pl.program_id(0)

    @pl.when(i == 0)
    def _():
        sums_acc[...] = jnp.zeros_like(sums_acc)
        cnts_acc[...] = jnp.zeros_like(cnts_acc)

    dinv = lax.rsqrt(degT_ref[:, 0:1] + degT_ref[:, 1:2] + 1.0)
    h2 = jax.nn.relu((acc_a_ref[...] + acc_b_ref[...] + g2_ref[...]) * dinv
                     + b2_ref[...])
    gid = lax.broadcasted_iota(jnp.int32, (_NBLK, _G), 1)
    oh = jnp.where(batch_ref[...] == gid, 1.0, 0.0)
    sums_acc[...] += lax.dot_general(
        oh, h2, (((0,), (0,)), ((), ())), preferred_element_type=_f32)
    cnts_acc[...] += lax.dot_general(
        oh, jnp.ones((_NBLK, _D), _f32), (((0,), (0,)), ((), ())),
        preferred_element_type=_f32)

    @pl.when(i == _NGRID - 1)
    def _():
        mean = sums_acc[...] / jnp.maximum(cnts_acc[...], 1.0)
        ho = jnp.dot(mean, Wl1_ref[...],
                     preferred_element_type=_f32) + bl1_ref[...]
        ax = jnp.dot(axp_ref[...], Wax_ref[...],
                     preferred_element_type=_f32) + bax_ref[...]
        o_ref[...] = (jnp.dot(ho, Wl2h_ref[...], preferred_element_type=_f32)
                      + jnp.dot(ax, Wl2a_ref[...],
                                preferred_element_type=_f32)
                      + bl2_ref[...])


def _deg_call(dstp):
    return _get_deg_kernel()(dstp)


def _scatter_call(g, srcp, dstp):
    return _get_scatter_kernel()(g, srcp, dstp)


_spec_full128 = pl.BlockSpec((_D, _D), lambda i: (0, 0))
_spec_row128 = pl.BlockSpec((1, _D), lambda i: (0, 0))
_spec_nblk = pl.BlockSpec((_NBLK, _D), lambda i: (i, 0))
_spec_deg = pl.BlockSpec((_NBLK, 2), lambda i: (i, 0))


def _k1a_call(x, W1):
    return pl.pallas_call(
        _k1a_body,
        grid=(_NGRID,),
        in_specs=[_spec_nblk, _spec_full128],
        out_specs=_spec_nblk,
        out_shape=jax.ShapeDtypeStruct((_N, _D), _f32),
    )(x, W1)


def _k1b_call(p1, degT):
    return pl.pallas_call(
        _k1b_body,
        grid=(_NGRID,),
        in_specs=[_spec_nblk, _spec_deg],
        out_specs=_spec_nblk,
        out_shape=jax.ShapeDtypeStruct((_N, _D), _f32),
    )(p1, degT)


def _k3_call(acc0, acc1, g1, degT, b1p, W2):
    return pl.pallas_call(
        _k3_body,
        grid=(_NGRID,),
        in_specs=[_spec_nblk, _spec_nblk, _spec_nblk, _spec_deg,
                  _spec_row128, _spec_full128],
        out_specs=_spec_nblk,
        out_shape=jax.ShapeDtypeStruct((_N, _D), _f32),
    )(acc0, acc1, g1, degT, b1p, W2)


def _k5_call(acc0, acc1, g2, degT, b2p, batch2d, axp, Wl1, bl1p, Waxp, baxp,
             Wl2h, Wl2a, bl2p):
    return pl.pallas_call(
        _k5_body,
        grid=(_NGRID,),
        in_specs=[_spec_nblk, _spec_nblk, _spec_nblk, _spec_deg,
                  _spec_row128,
                  pl.BlockSpec((_NBLK, 1), lambda i: (i, 0)),
                  pl.BlockSpec((_G, _D), lambda i: (0, 0)),
                  _spec_full128, _spec_row128, _spec_full128, _spec_row128,
                  _spec_full128, _spec_full128, _spec_row128],
        out_specs=pl.BlockSpec((_G, _D), lambda i: (0, 0)),
        out_shape=jax.ShapeDtypeStruct((_G, _D), _f32),
        scratch_shapes=[
            pltpu.VMEM((_G, _D), _f32),
            pltpu.VMEM((_G, _D), _f32),
        ],
    )(acc0, acc1, g2, degT, b2p, batch2d, axp, Wl1, bl1p, Waxp, baxp,
      Wl2h, Wl2a, bl2p)


def kernel(x, edge_index, batch, ax_data, W1, b1, W2, b2, Wl1, bl1,
           Wax, bax, Wl2, bl2):
    srcp = edge_index[0]
    dstp = edge_index[1]
    batch2d = batch.reshape(_N, 1)
    b1p = b1.reshape(1, _D)
    b2p = b2.reshape(1, _D)
    axp = jnp.pad(ax_data, ((0, 0), (0, _D - ax_data.shape[1])))
    Waxp = jnp.pad(Wax, ((0, _D - Wax.shape[0]), (0, _D - Wax.shape[1])))
    baxp = jnp.pad(bax, (0, _D - bax.shape[0])).reshape(1, _D)
    Wl2h = jnp.pad(Wl2[:_D], ((0, 0), (0, _D - Wl2.shape[1])))
    Wl2a = jnp.pad(Wl2[_D:], ((0, _D - (Wl2.shape[0] - _D)),
                              (0, _D - Wl2.shape[1])))
    bl2p = jnp.pad(bl2, (0, _D - bl2.shape[0])).reshape(1, _D)
    bl1p = bl1.reshape(1, _D)

    deg2 = _deg_call(dstp)
    p1 = _k1a_call(x, W1)
    degT = deg2.reshape(_NC, _NDEG).T
    g1 = _k1b_call(p1, degT)
    acc1a, acc1b = _scatter_call(g1, srcp, dstp)
    g2 = _k3_call(acc1a, acc1b, g1, degT, b1p, W2)
    acc2a, acc2b = _scatter_call(g2, srcp, dstp)
    out128 = _k5_call(acc2a, acc2b, g2, degT, b2p, batch2d, axp, Wl1, bl1p,
                      Waxp, baxp, Wl2h, Wl2a, bl2p)
    return out128[:, :Wl2.shape[1]]


# speedup vs baseline: 1.0469x; 1.0453x over previous
import functools

import jax
import jax.numpy as jnp
from jax import lax
from jax.experimental import pallas as pl
from jax.experimental.pallas import tpu as pltpu
from jax.experimental.pallas import tpu_sc as plsc

_N = 10000
_D = 128
_E = 320000
_G = 256
_NC = 2
_NS = 16
_TILES = _NC * _NS
_ERPT = 80
_EROWS = _TILES * _ERPT
_EPAD = _EROWS * 128
_NBUF = 2
_NACC = 10240
_STRIPE = _NACC // _NS
_NBLK = 1000
_NGRID = _N // _NBLK

_f32 = jnp.float32


def _zero_vmem_2d(ref, rows):
    def body(i, _):
        r = i // 8
        l = (i % 8) * 16
        ref[r, pl.ds(l, 16)] = jnp.zeros((16,), _f32)
        return 0
    lax.fori_loop(0, rows * 8, body, 0)


@functools.cache
def _get_deg_kernel():
    mesh = plsc.VectorSubcoreMesh(
        core_axis_name="c", subcore_axis_name="s",
        num_cores=_NC, num_subcores=_NS)
    return pl.kernel(
        _deg_body,
        out_type=jax.ShapeDtypeStruct((_NC * _NACC,), _f32),
        mesh=mesh,
        scratch_types=[
            pltpu.MemorySpace.VMEM_SHARED((_NACC,), _f32),
            pltpu.MemorySpace.VMEM((_ERPT, 128), jnp.int32),
            pltpu.MemorySpace.VMEM((128,), _f32),
            pltpu.MemorySpace.VMEM((_STRIPE,), _f32),
        ],
    )


def _deg_body(dst_hbm, out_hbm, acc_sh, dst_all, ones_v, zrow_v):
    c = lax.axis_index("c")
    s = lax.axis_index("s")
    base = (c * _NS + s) * _ERPT

    def zb(i, _):
        zrow_v[pl.ds(i * 16, 16)] = jnp.zeros((16,), _f32)
        return 0
    lax.fori_loop(0, _STRIPE // 16, zb, 0)

    def ob(i, _):
        ones_v[pl.ds(i * 16, 16)] = jnp.ones((16,), _f32)
        return 0
    lax.fori_loop(0, 8, ob, 0)

    pltpu.sync_copy(dst_hbm.at[pl.ds(base, _ERPT)], dst_all)
    pltpu.sync_copy(zrow_v, acc_sh.at[pl.ds(s * _STRIPE, _STRIPE)])
    plsc.subcore_barrier()

    def eloop(j, _):
        pltpu.sync_copy(ones_v, acc_sh.at[dst_all.at[j]], add=True)
        return 0
    lax.fori_loop(0, _ERPT, eloop, 0)

    plsc.subcore_barrier()
    pltpu.sync_copy(acc_sh.at[pl.ds(s * _STRIPE, _STRIPE)],
                    out_hbm.at[pl.ds(c * _NACC + s * _STRIPE, _STRIPE)])


@functools.cache
def _get_scatter_kernel():
    mesh = plsc.VectorSubcoreMesh(
        core_axis_name="c", subcore_axis_name="s",
        num_cores=_NC, num_subcores=_NS)
    return pl.kernel(
        _scatter_body,
        out_type=[jax.ShapeDtypeStruct((_NACC, _D), _f32),
                  jax.ShapeDtypeStruct((_NACC, _D), _f32)],
        mesh=mesh,
        scratch_types=[
            pltpu.MemorySpace.VMEM_SHARED((_NACC, _D), _f32),
            pltpu.MemorySpace.VMEM((_ERPT, 128), jnp.int32),
            pltpu.MemorySpace.VMEM((_NBUF, 128), jnp.int32),
            [pltpu.MemorySpace.VMEM((128, _D), _f32) for _ in range(_NBUF)],
            [pltpu.SemaphoreType.DMA for _ in range(_NBUF)],
            [pltpu.SemaphoreType.DMA for _ in range(_NBUF)],
        ],
    )


def _scatter_body(g_hbm, src_hbm, dst_hbm, out0_hbm, out1_hbm,
                  acc_sh, src_all, dstb, bufs, gsems, dsems):
    c = lax.axis_index("c")
    s = lax.axis_index("s")
    base = (c * _NS + s) * _ERPT

    pltpu.sync_copy(src_hbm.at[pl.ds(base, _ERPT)], src_all)

    with jax.named_scope("zero_acc"):
        _zero_vmem_2d(bufs[0], 128)
        for k in range(_STRIPE // 128):
            pltpu.sync_copy(bufs[0],
                            acc_sh.at[pl.ds(s * _STRIPE + k * 128, 128)])

    for b in range(_NBUF):
        pltpu.async_copy(dst_hbm.at[base + b], dstb.at[b], dsems[b])
        pltpu.async_copy(g_hbm.at[src_all.at[b]], bufs[b], gsems[b])
    plsc.subcore_barrier()

    with jax.named_scope("edge_loop"):
        def outer(i, _):
            j0 = i * _NBUF
            for b in range(_NBUF):
                j = j0 + b
                pltpu.make_async_copy(g_hbm.at[src_all.at[j]],
                                      bufs[b], gsems[b]).wait()
                pltpu.make_async_copy(dst_hbm.at[base + j],
                                      dstb.at[b], dsems[b]).wait()
                pltpu.sync_copy(bufs[b], acc_sh.at[dstb.at[b]], add=True)
                jn = jnp.minimum(j + _NBUF, _ERPT - 1)
                pltpu.async_copy(dst_hbm.at[base + jn], dstb.at[b], dsems[b])
                pltpu.async_copy(g_hbm.at[src_all.at[jn]], bufs[b], gsems[b])
            return 0
        lax.fori_loop(0, _ERPT // _NBUF, outer, 0)

        for b in range(_NBUF):
            pltpu.make_async_copy(g_hbm.at[src_all.at[_ERPT - 1]],
                                  bufs[b], gsems[b]).wait()
            pltpu.make_async_copy(dst_hbm.at[base + _ERPT - 1],
                                  dstb.at[b], dsems[b]).wait()

    with jax.named_scope("end_barrier"):
        plsc.subcore_barrier()
    with jax.named_scope("copy_out"):
        @pl.when(c == 0)
        def _():
            pltpu.sync_copy(acc_sh.at[pl.ds(s * _STRIPE, _STRIPE)],
                            out0_hbm.at[pl.ds(s * _STRIPE, _STRIPE)])

        @pl.when(c == 1)
        def _():
            pltpu.sync_copy(acc_sh.at[pl.ds(s * _STRIPE, _STRIPE)],
                            out1_hbm.at[pl.ds(s * _STRIPE, _STRIPE)])


def _k1a_body(x_ref, W1_ref, o_ref):
    o_ref[...] = jnp.dot(x_ref[...], W1_ref[...],
                         preferred_element_type=_f32)


def _k1b_body(p_ref, degT_ref, o_ref):
    dinv = lax.rsqrt(degT_ref[:, 0:1] + degT_ref[:, 1:2] + 1.0)
    o_ref[...] = p_ref[...] * dinv


def _k3_body(acc_a_ref, acc_b_ref, g1_ref, degT_ref, b1_ref, W2_ref, o_ref):
    dinv = lax.rsqrt(degT_ref[:, 0:1] + degT_ref[:, 1:2] + 1.0)
    h1 = jax.nn.relu((acc_a_ref[...] + acc_b_ref[...] + g1_ref[...]) * dinv
                     + b1_ref[...])
    o_ref[...] = jnp.dot(h1, W2_ref[...], preferred_element_type=_f32) * dinv


def _k5_body(acc_a_ref, acc_b_ref, g2_ref, degT_ref, b2_ref, batch_ref,
             axp_ref, Wl1_ref, bl1_ref, Wax_ref, bax_ref,
             Wl2h_ref, Wl2a_ref, bl2_ref, o_ref, sums_acc, cnts_acc):
    i = pl.program_id(0)

    @pl.when(i == 0)
    def _():
        sums_acc[...] = jnp.zeros_like(sums_acc)
        cnts_acc[...] = jnp.zeros_like(cnts_acc)

    dinv = lax.rsqrt(degT_ref[:, 0:1] + degT_ref[:, 1:2] + 1.0)
    h2 = jax.nn.relu((acc_a_ref[...] + acc_b_ref[...] + g2_ref[...]) * dinv
                     + b2_ref[...])
    gid = lax.broadcasted_iota(jnp.int32, (_NBLK, _G), 1)
    oh = jnp.where(batch_ref[...] == gid, 1.0, 0.0)
    sums_acc[...] += lax.dot_general(
        oh, h2, (((0,), (0,)), ((), ())), preferred_element_type=_f32)
    cnts_acc[...] += lax.dot_general(
        oh, jnp.ones((_NBLK, _D), _f32), (((0,), (0,)), ((), ())),
        preferred_element_type=_f32)

    @pl.when(i == _NGRID - 1)
    def _():
        mean = sums_acc[...] / jnp.maximum(cnts_acc[...], 1.0)
        ho = jnp.dot(mean, Wl1_ref[...],
                     preferred_element_type=_f32) + bl1_ref[...]
        ax = jnp.dot(axp_ref[...], Wax_ref[...],
                     preferred_element_type=_f32) + bax_ref[...]
        o_ref[...] = (jnp.dot(ho, Wl2h_ref[...], preferred_element_type=_f32)
                      + jnp.dot(ax, Wl2a_ref[...],
                                preferred_element_type=_f32)
                      + bl2_ref[...])


def _deg_call(dstp):
    return _get_deg_kernel()(dstp)


def _scatter_call(g, srcp, dstp):
    return _get_scatter_kernel()(g, srcp, dstp)


_spec_full128 = pl.BlockSpec((_D, _D), lambda i: (0, 0))
_spec_row128 = pl.BlockSpec((1, _D), lambda i: (0, 0))
_spec_nblk = pl.BlockSpec((_NBLK, _D), lambda i: (i, 0))
_spec_deg = pl.BlockSpec((_NBLK, 2), lambda i: (i, 0))


def _k1a_call(x, W1):
    return pl.pallas_call(
        _k1a_body,
        grid=(_NGRID,),
        in_specs=[_spec_nblk, _spec_full128],
        out_specs=_spec_nblk,
        out_shape=jax.ShapeDtypeStruct((_N, _D), _f32),
    )(x, W1)


def _k1b_call(p1, degT):
    return pl.pallas_call(
        _k1b_body,
        grid=(_NGRID,),
        in_specs=[_spec_nblk, _spec_deg],
        out_specs=_spec_nblk,
        out_shape=jax.ShapeDtypeStruct((_N, _D), _f32),
    )(p1, degT)


def _k3_call(acc0, acc1, g1, degT, b1p, W2):
    return pl.pallas_call(
        _k3_body,
        grid=(_NGRID,),
        in_specs=[_spec_nblk, _spec_nblk, _spec_nblk, _spec_deg,
                  _spec_row128, _spec_full128],
        out_specs=_spec_nblk,
        out_shape=jax.ShapeDtypeStruct((_N, _D), _f32),
    )(acc0, acc1, g1, degT, b1p, W2)


def _k5_call(acc0, acc1, g2, degT, b2p, batch2d, axp, Wl1, bl1p, Waxp, baxp,
             Wl2h, Wl2a, bl2p):
    return pl.pallas_call(
        _k5_body,
        grid=(_NGRID,),
        in_specs=[_spec_nblk, _spec_nblk, _spec_nblk, _spec_deg,
                  _spec_row128,
                  pl.BlockSpec((_NBLK, 1), lambda i: (i, 0)),
                  pl.BlockSpec((_G, _D), lambda i: (0, 0)),
                  _spec_full128, _spec_row128, _spec_full128, _spec_row128,
                  _spec_full128, _spec_full128, _spec_row128],
        out_specs=pl.BlockSpec((_G, _D), lambda i: (0, 0)),
        out_shape=jax.ShapeDtypeStruct((_G, _D), _f32),
        scratch_shapes=[
            pltpu.VMEM((_G, _D), _f32),
            pltpu.VMEM((_G, _D), _f32),
        ],
    )(acc0, acc1, g2, degT, b2p, batch2d, axp, Wl1, bl1p, Waxp, baxp,
      Wl2h, Wl2a, bl2p)


def kernel(x, edge_index, batch, ax_data, W1, b1, W2, b2, Wl1, bl1,
           Wax, bax, Wl2, bl2):
    pad_e = _EPAD - _E
    pad_i = jnp.arange(pad_e, dtype=jnp.int32) % (_NACC - _N)
    srcp = jnp.concatenate([edge_index[0], pad_i]).reshape(_EROWS, 128)
    dstp = jnp.concatenate([edge_index[1], _N + pad_i]).reshape(_EROWS, 128)
    batch2d = batch.reshape(_N, 1)
    b1p = b1.reshape(1, _D)
    b2p = b2.reshape(1, _D)
    axp = jnp.pad(ax_data, ((0, 0), (0, _D - ax_data.shape[1])))
    Waxp = jnp.pad(Wax, ((0, _D - Wax.shape[0]), (0, _D - Wax.shape[1])))
    baxp = jnp.pad(bax, (0, _D - bax.shape[0])).reshape(1, _D)
    Wl2h = jnp.pad(Wl2[:_D], ((0, 0), (0, _D - Wl2.shape[1])))
    Wl2a = jnp.pad(Wl2[_D:], ((0, _D - (Wl2.shape[0] - _D)),
                              (0, _D - Wl2.shape[1])))
    bl2p = jnp.pad(bl2, (0, _D - bl2.shape[0])).reshape(1, _D)
    bl1p = bl1.reshape(1, _D)

    deg2 = _deg_call(dstp)
    p1 = _k1a_call(x, W1)
    degT = deg2.reshape(_NC, _NACC).T
    g1 = _k1b_call(p1, degT)
    acc1a, acc1b = _scatter_call(g1, srcp, dstp)
    g2 = _k3_call(acc1a, acc1b, g1, degT, b1p, W2)
    acc2a, acc2b = _scatter_call(g2, srcp, dstp)
    out128 = _k5_call(acc2a, acc2b, g2, degT, b2p, batch2d, axp, Wl1, bl1p,
                      Waxp, baxp, Wl2h, Wl2a, bl2p)
    return out128[:, :Wl2.shape[1]]
